# Initial kernel scaffold; baseline (speedup 1.0000x reference)
import jax
import jax.numpy as jnp
from jax.experimental import pallas as pl


def kernel(left_chunks, right_chunks, constr):
    raise NotImplementedError("write your pallas kernel here")


# speedup vs baseline: 2.3642x; 2.3642x over previous
import functools

import jax
import jax.numpy as jnp
from jax import lax
from jax.experimental import pallas as pl
from jax.experimental.pallas import tpu as pltpu
from jax.experimental.pallas import tpu_sc as plsc

_NUM_CORES = 2
_NUM_SUBCORES = 16
_NW = _NUM_CORES * _NUM_SUBCORES


def _vgather(x, idx):
    return x.at[idx].get(mode="promise_in_bounds")


def _pick_chunk(k: int) -> int:
    for c in range(min(4000, k), 7, -1):
        if k % c == 0 and c % 8 == 0:
            return c
    return k


@functools.lru_cache(maxsize=None)
def _make_zero_fill(n: int):
    blk = 524288
    while n % blk != 0:
        blk //= 2
    grid = n // blk

    def body(o_ref):
        o_ref[...] = jnp.zeros_like(o_ref)

    return pl.pallas_call(
        body,
        grid=(grid,),
        out_specs=pl.BlockSpec((blk,), lambda i: (i,)),
        out_shape=jax.ShapeDtypeStruct((n,), jnp.float32),
    )


@functools.lru_cache(maxsize=None)
def _make_scatter(k: int, n: int, r: int):
    c = _pick_chunk(k)
    nch = k // c
    tmax = -(-nch // _NW)

    mesh = plsc.VectorSubcoreMesh(
        core_axis_name="c", subcore_axis_name="s"
    )

    @functools.partial(
        pl.kernel,
        mesh=mesh,
        out_type=(),
        scratch_types=[
            pltpu.VMEM((2 * c,), jnp.int32),
            pltpu.VMEM((c,), jnp.int32),
            pltpu.VMEM((c,), jnp.float32),
        ],
    )
    def scatter(pairs_hbm, ones_hbm, mask_hbm, pair_v, idx_v, ones_v):
        wid = lax.axis_index("s") * _NUM_CORES + lax.axis_index("c")
        pltpu.sync_copy(ones_hbm, ones_v)
        lane = lax.iota(jnp.int32, 16)

        def chunk_body(t, carry):
            cid = wid + _NW * t

            @pl.when(cid < nch)
            def _():
                base = pl.multiple_of(cid * (2 * c), 8)
                pltpu.sync_copy(pairs_hbm.at[pl.ds(base, 2 * c)], pair_v)

                def flat_body(j, carry2):
                    v0 = pair_v[pl.ds(j * 32, 16)]
                    v1 = pair_v[pl.ds(j * 32 + 16, 16)]
                    g = (lane * 2) % 16
                    lo = lane < 8
                    s1 = jnp.where(lo, _vgather(v0, g), _vgather(v1, g))
                    s2 = jnp.where(
                        lo, _vgather(v0, g + 1), _vgather(v1, g + 1)
                    )
                    idx_v[pl.ds(j * 16, 16)] = s1 * r + s2
                    return carry2

                lax.fori_loop(0, c // 16, flat_body, 0)
                pltpu.sync_copy(ones_v, mask_hbm.at[idx_v])

            return carry

        lax.fori_loop(0, tmax, chunk_body, 0)

    return scatter


def kernel(left_chunks, right_chunks, constr):
    l = left_chunks.shape[0]
    r = right_chunks.shape[0]
    k = constr.shape[0]
    n = l * r

    pairs = constr.reshape(2 * k)
    zeroed = _make_zero_fill(n)()
    ones = jnp.ones((_pick_chunk(k),), jnp.float32)

    mask_ref = jax.new_ref(zeroed)
    _make_scatter(k, n, r)(pairs, ones, mask_ref)
    return mask_ref[...].reshape(l, r)
